# Initial kernel scaffold; baseline (speedup 1.0000x reference)
import jax
import jax.numpy as jnp
from jax.experimental import pallas as pl


def kernel(embed, edge_index, edge_values):
    raise NotImplementedError("write your pallas kernel here")


# speedup vs baseline: 3.2200x; 3.2200x over previous
import functools

import jax
import jax.numpy as jnp
from jax import lax
from jax.experimental import pallas as pl
from jax.experimental.pallas import tpu as pltpu
from jax.experimental.pallas import tpu_sc as plsc

N_NODES = 10000
N_EDGES = 320000
D_FEAT = 128
N_HOPS = 3
NC = 2
NS = 16
DH = D_FEAT // NC
W = 64
NBLK = N_EDGES // W
G = 4
BLK_PER_TILE = NBLK // NS
BLK_REM = NBLK % NS
MAX_BLK = BLK_PER_TILE + 1
N_GROUPS = (MAX_BLK + G - 1) // G
ROWS_PER_TILE = N_NODES // NS
RCHUNK = 125


def _body(embed_hbm, col_hbm, row_hbm, wgt_hbm, out_hbm,
          acc, gbuf, cbig, rbig, wbig, zbuf, gsem):
  cid = lax.axis_index("c")
  sid = lax.axis_index("s")

  def _zero_row(i, _):
    for q in range(DH // 16):
      zbuf[i, pl.ds(q * 16, 16)] = jnp.zeros((16,), jnp.float32)
    return 0
  lax.fori_loop(0, RCHUNK, _zero_row, 0)

  base_blk = sid * BLK_PER_TILE + jnp.minimum(sid, BLK_REM)
  n_blk = BLK_PER_TILE + jnp.where(sid < BLK_REM, 1, 0)
  ld_base = jnp.minimum(base_blk, NBLK - MAX_BLK)

  for kk in range(ROWS_PER_TILE // RCHUNK):
    r0 = sid * ROWS_PER_TILE + kk * RCHUNK
    pltpu.sync_copy(zbuf, acc.at[pl.ds(r0, RCHUNK)])

  pltpu.sync_copy(col_hbm.at[pl.ds(ld_base, MAX_BLK)], cbig)
  pltpu.sync_copy(row_hbm.at[pl.ds(ld_base, MAX_BLK)], rbig)
  pltpu.sync_copy(wgt_hbm.at[pl.ds(ld_base, MAX_BLK)], wbig)
  off_blk = base_blk - ld_base

  plsc.subcore_barrier()

  for hop in range(N_HOPS):
    tbl = embed_hbm if hop == 0 else out_hbm.at[hop - 1]

    def _group(g, _, tbl=tbl):
      for b in range(G):
        jj = g * G + b
        @pl.when(jj < n_blk)
        def _():
          pltpu.async_copy(tbl.at[cid].at[cbig.at[off_blk + jj]],
                           gbuf.at[pl.ds(b * W, W)], gsem.at[b])
      for b in range(G):
        jj = g * G + b

        @pl.when(jj < n_blk)
        def _():
          pltpu.make_async_copy(tbl.at[cid].at[cbig.at[off_blk + jj]],
                                gbuf.at[pl.ds(b * W, W)], gsem.at[b]).wait()

          def _scale16(k, _):
            wv = wbig[off_blk + jj, pl.ds(k * 16, 16)]
            for l in range(16):
              e = b * W + k * 16 + l
              ws = jnp.broadcast_to(wv[l], (16,))
              for q in range(DH // 16):
                sl = pl.ds(q * 16, 16)
                gbuf[e, sl] = gbuf[e, sl] * ws
            return 0
          lax.fori_loop(0, W // 16, _scale16, 0)

          pltpu.sync_copy(gbuf.at[pl.ds(b * W, W)], acc.at[rbig.at[off_blk + jj]],
                          add=True)
      return 0

    lax.fori_loop(0, N_GROUPS, _group, 0)

    plsc.subcore_barrier()

    for kk in range(ROWS_PER_TILE // RCHUNK):
      r0 = sid * ROWS_PER_TILE + kk * RCHUNK
      stg = gbuf.at[pl.ds(0, RCHUNK)]
      pltpu.sync_copy(acc.at[pl.ds(r0, RCHUNK)], stg)
      pltpu.sync_copy(stg, out_hbm.at[hop].at[cid].at[pl.ds(r0, RCHUNK)])
      if hop < N_HOPS - 1:
        pltpu.sync_copy(zbuf, acc.at[pl.ds(r0, RCHUNK)])

    plsc.subcore_barrier()


def kernel(embed, edge_index, edge_values):
  col2 = edge_index[1].reshape(NBLK, W)
  row2 = edge_index[0].reshape(NBLK, W)
  wgt2 = edge_values.reshape(NBLK, W)
  embed_split = embed.reshape(N_NODES, NC, DH).transpose(1, 0, 2)

  mesh = plsc.VectorSubcoreMesh(core_axis_name="c", subcore_axis_name="s")
  hops_split = pl.kernel(
      _body,
      out_type=jax.ShapeDtypeStruct((N_HOPS, NC, N_NODES, DH), jnp.float32),
      mesh=mesh,
      compiler_params=pltpu.CompilerParams(use_tc_tiling_on_sc=False),
      scratch_types=[
          pltpu.VMEM_SHARED((N_NODES, DH), jnp.float32),
          pltpu.VMEM((G * W, DH), jnp.float32),
          pltpu.VMEM((MAX_BLK, W), jnp.int32),
          pltpu.VMEM((MAX_BLK, W), jnp.int32),
          pltpu.VMEM((MAX_BLK, W), jnp.float32),
          pltpu.VMEM((RCHUNK, DH), jnp.float32),
          pltpu.SemaphoreType.DMA((G,)),
      ],
  )(embed_split, col2, row2, wgt2)

  hops = hops_split.transpose(2, 0, 1, 3).reshape(N_NODES, N_HOPS, D_FEAT)
  return jnp.concatenate([embed[:, None, :], hops], axis=1)


# speedup vs baseline: 5.5906x; 1.7362x over previous
import functools

import jax
import jax.numpy as jnp
from jax import lax
from jax.experimental import pallas as pl
from jax.experimental.pallas import tpu as pltpu
from jax.experimental.pallas import tpu_sc as plsc

N_NODES = 10000
N_EDGES = 320000
D_FEAT = 128
N_HOPS = 3
NC = 2
NS = 16
DH = D_FEAT // NC
W = 128
NBLK = N_EDGES // W
G = 4
BLK_PER_TILE = NBLK // NS
BLK_REM = NBLK % NS
MAX_BLK = BLK_PER_TILE + 1
N_GROUPS = (MAX_BLK + G - 1) // G
ROWS_PER_TILE = N_NODES // NS
RCHUNK = 125


def _body(embed_hbm, pk_hbm, wgt_hbm, out_hbm,
          acc, gbuf, pbig, wbig, colw, roww, zbuf, gsem):
  cid = lax.axis_index("c")
  sid = lax.axis_index("s")

  def _zero_row(i, _):
    for q in range(DH // 16):
      zbuf[i, pl.ds(q * 16, 16)] = jnp.zeros((16,), jnp.float32)
    return 0
  lax.fori_loop(0, RCHUNK, _zero_row, 0)

  base_blk = sid * BLK_PER_TILE + jnp.minimum(sid, BLK_REM)
  n_blk = BLK_PER_TILE + jnp.where(sid < BLK_REM, 1, 0)
  ld_base = jnp.minimum(base_blk, NBLK - MAX_BLK)
  off = base_blk - ld_base

  for kk in range(ROWS_PER_TILE // RCHUNK):
    r0 = sid * ROWS_PER_TILE + kk * RCHUNK
    pltpu.sync_copy(zbuf, acc.at[pl.ds(r0, RCHUNK)])

  pltpu.sync_copy(pk_hbm.at[pl.ds(ld_base, MAX_BLK)], pbig)
  pltpu.sync_copy(wgt_hbm.at[pl.ds(ld_base, MAX_BLK)], wbig)

  plsc.subcore_barrier()

  def _unpack(jj, b):
    def _u16(k, _):
      p = pbig[off + jj, pl.ds(k * 16, 16)]
      colw[b, pl.ds(k * 16, 16)] = jnp.bitwise_and(p, 0xFFFF)
      roww[b, pl.ds(k * 16, 16)] = lax.shift_right_logical(p, 16)
      return 0
    lax.fori_loop(0, W // 16, _u16, 0)

  for hop in range(N_HOPS):
    tbl = embed_hbm if hop == 0 else out_hbm.at[hop - 1]

    for b in range(G):
      _unpack(b, b)
      pltpu.async_copy(tbl.at[cid].at[colw.at[b]],
                       gbuf.at[pl.ds(b * W, W)], gsem.at[b])

    def _group(g, _, tbl=tbl):
      for b in range(G):
        jj = g * G + b

        @pl.when(jj < n_blk)
        def _():
          pltpu.make_async_copy(tbl.at[cid].at[colw.at[b]],
                                gbuf.at[pl.ds(b * W, W)], gsem.at[b]).wait()

          def _scale16(k, _):
            wv = wbig[off + jj, pl.ds(k * 16, 16)]
            for l in range(16):
              e = b * W + k * 16 + l
              ws = jnp.broadcast_to(wv[l], (16,))
              for q in range(DH // 16):
                sl = pl.ds(q * 16, 16)
                gbuf[e, sl] = gbuf[e, sl] * ws
            return 0
          lax.fori_loop(0, W // 16, _scale16, 0)

          pltpu.sync_copy(gbuf.at[pl.ds(b * W, W)], acc.at[roww.at[b]],
                          add=True)

          @pl.when(jj + G < n_blk)
          def _():
            _unpack(jj + G, b)
            pltpu.async_copy(tbl.at[cid].at[colw.at[b]],
                             gbuf.at[pl.ds(b * W, W)], gsem.at[b])
      return 0

    lax.fori_loop(0, N_GROUPS, _group, 0)

    plsc.subcore_barrier()

    for kk in range(ROWS_PER_TILE // RCHUNK):
      r0 = sid * ROWS_PER_TILE + kk * RCHUNK
      stg = gbuf.at[pl.ds(0, RCHUNK)]
      pltpu.sync_copy(acc.at[pl.ds(r0, RCHUNK)], stg)
      pltpu.sync_copy(stg, out_hbm.at[hop].at[cid].at[pl.ds(r0, RCHUNK)])
      if hop < N_HOPS - 1:
        pltpu.sync_copy(zbuf, acc.at[pl.ds(r0, RCHUNK)])

    plsc.subcore_barrier()


def kernel(embed, edge_index, edge_values):
  packed = (edge_index[0] * 65536 + edge_index[1]).reshape(NBLK, W)
  wgt2 = edge_values.reshape(NBLK, W)
  embed_split = embed.reshape(N_NODES, NC, DH).transpose(1, 0, 2)

  mesh = plsc.VectorSubcoreMesh(core_axis_name="c", subcore_axis_name="s")
  hops_split = pl.kernel(
      _body,
      out_type=jax.ShapeDtypeStruct((N_HOPS, NC, N_NODES, DH), jnp.float32),
      mesh=mesh,
      compiler_params=pltpu.CompilerParams(use_tc_tiling_on_sc=False),
      scratch_types=[
          pltpu.VMEM_SHARED((N_NODES, DH), jnp.float32),
          pltpu.VMEM((G * W, DH), jnp.float32),
          pltpu.VMEM((MAX_BLK, W), jnp.int32),
          pltpu.VMEM((MAX_BLK, W), jnp.float32),
          pltpu.VMEM((G, W), jnp.int32),
          pltpu.VMEM((G, W), jnp.int32),
          pltpu.VMEM((RCHUNK, DH), jnp.float32),
          pltpu.SemaphoreType.DMA((G,)),
      ],
  )(embed_split, packed, wgt2)

  hops = hops_split.transpose(2, 0, 1, 3).reshape(N_NODES, N_HOPS, D_FEAT)
  return jnp.concatenate([embed[:, None, :], hops], axis=1)


# speedup vs baseline: 8.6259x; 1.5429x over previous
import functools

import jax
import jax.numpy as jnp
from jax import lax
from jax.experimental import pallas as pl
from jax.experimental.pallas import tpu as pltpu
from jax.experimental.pallas import tpu_sc as plsc

N_NODES = 10000
N_EDGES = 320000
D_FEAT = 128
N_HOPS = 3
NC = 2
NS = 16
DH = D_FEAT // NC
W = 128
NBLK = N_EDGES // W
G = 4
BLK_PER_TILE = NBLK // NS
BLK_REM = NBLK % NS
MAX_BLK = BLK_PER_TILE + 1
N_GROUPS = (MAX_BLK + G - 1) // G
ROWS_PER_TILE = N_NODES // NS
RCHUNK = 125


def _body(embed_hbm, pk_hbm, wgt_hbm, out_hbm,
          acc, gbuf, pbig, wbig, colw, roww, zbuf, gsem):
  cid = lax.axis_index("c")
  sid = lax.axis_index("s")

  def _zero_row(i, _):
    for q in range(DH // 16):
      zbuf[i, pl.ds(q * 16, 16)] = jnp.zeros((16,), jnp.float32)
    return 0
  lax.fori_loop(0, RCHUNK, _zero_row, 0)

  base_blk = sid * BLK_PER_TILE + jnp.minimum(sid, BLK_REM)
  n_blk = BLK_PER_TILE + jnp.where(sid < BLK_REM, 1, 0)
  ld_base = jnp.minimum(base_blk, NBLK - MAX_BLK)
  off = base_blk - ld_base

  for kk in range(ROWS_PER_TILE // RCHUNK):
    r0 = sid * ROWS_PER_TILE + kk * RCHUNK
    pltpu.sync_copy(zbuf, acc.at[pl.ds(r0, RCHUNK)])

  pltpu.sync_copy(pk_hbm.at[pl.ds(ld_base, MAX_BLK)], pbig)
  pltpu.sync_copy(wgt_hbm.at[pl.ds(ld_base, MAX_BLK)], wbig)

  plsc.subcore_barrier()

  def _unpack(jj, b):
    def _u16(k, _):
      p = pbig[off + jj, pl.ds(k * 16, 16)]
      colw[b, pl.ds(k * 16, 16)] = jnp.bitwise_and(p, 0xFFFF)
      roww[b, pl.ds(k * 16, 16)] = lax.shift_right_logical(p, 16)
      return 0
    lax.fori_loop(0, W // 16, _u16, 0)

  for hop in range(N_HOPS):
    tbl = embed_hbm if hop == 0 else out_hbm.at[hop - 1]

    for b in range(G):
      _unpack(b, b)
      pltpu.async_copy(tbl.at[cid].at[colw.at[b]],
                       gbuf.at[pl.ds(b * W, W)], gsem.at[b])

    def _group(g, _, tbl=tbl):
      for b in range(G):
        jj = g * G + b

        @pl.when(jj < n_blk)
        def _():
          pltpu.make_async_copy(tbl.at[cid].at[colw.at[b]],
                                gbuf.at[pl.ds(b * W, W)], gsem.at[b]).wait()

          @plsc.parallel_loop(0, W // 16, unroll=2)
          def _scale16(k):
            wv = wbig[off + jj, pl.ds(k * 16, 16)]
            for l4 in range(4):
              vals, wss = [], []
              for l in range(4):
                e = b * W + k * 16 + l4 * 4 + l
                wss.append(jnp.broadcast_to(wv[l4 * 4 + l], (16,)))
                vals.append([gbuf[e, pl.ds(q * 16, 16)]
                             for q in range(DH // 16)])
              for l in range(4):
                e = b * W + k * 16 + l4 * 4 + l
                for q in range(DH // 16):
                  gbuf[e, pl.ds(q * 16, 16)] = vals[l][q] * wss[l]

          pltpu.sync_copy(gbuf.at[pl.ds(b * W, W)], acc.at[roww.at[b]],
                          add=True)

          @pl.when(jj + G < n_blk)
          def _():
            _unpack(jj + G, b)
            pltpu.async_copy(tbl.at[cid].at[colw.at[b]],
                             gbuf.at[pl.ds(b * W, W)], gsem.at[b])
      return 0

    lax.fori_loop(0, N_GROUPS, _group, 0)

    plsc.subcore_barrier()

    for kk in range(ROWS_PER_TILE // RCHUNK):
      r0 = sid * ROWS_PER_TILE + kk * RCHUNK
      stg = gbuf.at[pl.ds(0, RCHUNK)]
      pltpu.sync_copy(acc.at[pl.ds(r0, RCHUNK)], stg)
      pltpu.sync_copy(stg, out_hbm.at[hop].at[cid].at[pl.ds(r0, RCHUNK)])
      if hop < N_HOPS - 1:
        pltpu.sync_copy(zbuf, acc.at[pl.ds(r0, RCHUNK)])

    plsc.subcore_barrier()


def kernel(embed, edge_index, edge_values):
  packed = (edge_index[0] * 65536 + edge_index[1]).reshape(NBLK, W)
  wgt2 = edge_values.reshape(NBLK, W)
  embed_split = embed.reshape(N_NODES, NC, DH).transpose(1, 0, 2)

  mesh = plsc.VectorSubcoreMesh(core_axis_name="c", subcore_axis_name="s")
  hops_split = pl.kernel(
      _body,
      out_type=jax.ShapeDtypeStruct((N_HOPS, NC, N_NODES, DH), jnp.float32),
      mesh=mesh,
      compiler_params=pltpu.CompilerParams(use_tc_tiling_on_sc=False),
      scratch_types=[
          pltpu.VMEM_SHARED((N_NODES, DH), jnp.float32),
          pltpu.VMEM((G * W, DH), jnp.float32),
          pltpu.VMEM((MAX_BLK, W), jnp.int32),
          pltpu.VMEM((MAX_BLK, W), jnp.float32),
          pltpu.VMEM((G, W), jnp.int32),
          pltpu.VMEM((G, W), jnp.int32),
          pltpu.VMEM((RCHUNK, DH), jnp.float32),
          pltpu.SemaphoreType.DMA((G,)),
      ],
  )(embed_split, packed, wgt2)

  hops = hops_split.transpose(2, 0, 1, 3).reshape(N_NODES, N_HOPS, D_FEAT)
  return jnp.concatenate([embed[:, None, :], hops], axis=1)


# speedup vs baseline: 9.7071x; 1.1253x over previous
import functools

import jax
import jax.numpy as jnp
from jax import lax
from jax.experimental import pallas as pl
from jax.experimental.pallas import tpu as pltpu
from jax.experimental.pallas import tpu_sc as plsc

N_NODES = 10000
N_EDGES = 320000
D_FEAT = 128
N_HOPS = 3
NC = 2
NS = 16
DH = D_FEAT // NC
W = 128
NBLK = N_EDGES // W
G = 4
BLK_PER_TILE = NBLK // NS
BLK_REM = NBLK % NS
MAX_BLK = BLK_PER_TILE + 1
N_GROUPS = (MAX_BLK + G - 1) // G
ROWS_PER_TILE = N_NODES // NS
RCHUNK = 125


def _body(embed_hbm, pk_hbm, wgt_hbm, out_hbm, tbl_hbm,
          acc, gbuf, pbig, wbig, colw, roww, zbuf, gsem):
  cid = lax.axis_index("c")
  sid = lax.axis_index("s")

  def _zero_row(i, _):
    for q in range(DH // 16):
      zbuf[i, pl.ds(q * 16, 16)] = jnp.zeros((16,), jnp.float32)
    return 0
  lax.fori_loop(0, RCHUNK, _zero_row, 0)

  base_blk = sid * BLK_PER_TILE + jnp.minimum(sid, BLK_REM)
  n_blk = BLK_PER_TILE + jnp.where(sid < BLK_REM, 1, 0)
  ld_base = jnp.minimum(base_blk, NBLK - MAX_BLK)
  off = base_blk - ld_base

  for kk in range(ROWS_PER_TILE // RCHUNK):
    r0 = sid * ROWS_PER_TILE + kk * RCHUNK
    pltpu.sync_copy(zbuf, acc.at[pl.ds(r0, RCHUNK)])

  pltpu.sync_copy(pk_hbm.at[pl.ds(ld_base, MAX_BLK)], pbig)
  pltpu.sync_copy(wgt_hbm.at[pl.ds(ld_base, MAX_BLK)], wbig)

  plsc.subcore_barrier()

  def _unpack(jj, b):
    def _u16(k, _):
      p = pbig[off + jj, pl.ds(k * 16, 16)]
      colw[b, pl.ds(k * 16, 16)] = jnp.bitwise_and(p, 0xFFFF)
      roww[b, pl.ds(k * 16, 16)] = lax.shift_right_logical(p, 16)
      return 0
    lax.fori_loop(0, W // 16, _u16, 0)

  for hop in range(N_HOPS):
    tbl = embed_hbm if hop == 0 else tbl_hbm.at[hop - 1]

    for b in range(G):
      _unpack(b, b)
      pltpu.async_copy(tbl.at[cid].at[colw.at[b]],
                       gbuf.at[pl.ds(b * W, W)], gsem.at[b])

    def _group(g, _, tbl=tbl):
      for b in range(G):
        jj = g * G + b

        @pl.when(jj < n_blk)
        def _():
          pltpu.make_async_copy(tbl.at[cid].at[colw.at[b]],
                                gbuf.at[pl.ds(b * W, W)], gsem.at[b]).wait()

          @plsc.parallel_loop(0, W // 16, unroll=2)
          def _scale16(k):
            wv = wbig[off + jj, pl.ds(k * 16, 16)]
            for l4 in range(4):
              vals, wss = [], []
              for l in range(4):
                e = b * W + k * 16 + l4 * 4 + l
                wss.append(jnp.broadcast_to(wv[l4 * 4 + l], (16,)))
                vals.append([gbuf[e, pl.ds(q * 16, 16)]
                             for q in range(DH // 16)])
              for l in range(4):
                e = b * W + k * 16 + l4 * 4 + l
                for q in range(DH // 16):
                  gbuf[e, pl.ds(q * 16, 16)] = vals[l][q] * wss[l]

          pltpu.sync_copy(gbuf.at[pl.ds(b * W, W)], acc.at[roww.at[b]],
                          add=True)

          @pl.when(jj + G < n_blk)
          def _():
            _unpack(jj + G, b)
            pltpu.async_copy(tbl.at[cid].at[colw.at[b]],
                             gbuf.at[pl.ds(b * W, W)], gsem.at[b])
      return 0

    lax.fori_loop(0, N_GROUPS, _group, 0)

    plsc.subcore_barrier()

    for kk in range(ROWS_PER_TILE // RCHUNK):
      r0 = sid * ROWS_PER_TILE + kk * RCHUNK
      stg = gbuf.at[pl.ds(0, RCHUNK)]
      pltpu.sync_copy(acc.at[pl.ds(r0, RCHUNK)], stg)
      pltpu.sync_copy(
          stg, out_hbm.at[pl.ds(r0, RCHUNK), hop + 1, pl.ds(cid * DH, DH)])
      if hop < N_HOPS - 1:
        pltpu.sync_copy(stg, tbl_hbm.at[hop].at[cid].at[pl.ds(r0, RCHUNK)])
        pltpu.sync_copy(zbuf, acc.at[pl.ds(r0, RCHUNK)])
      if hop == 0:
        stg2 = gbuf.at[pl.ds(RCHUNK, RCHUNK)]
        pltpu.sync_copy(embed_hbm.at[cid].at[pl.ds(r0, RCHUNK)], stg2)
        pltpu.sync_copy(
            stg2, out_hbm.at[pl.ds(r0, RCHUNK), 0, pl.ds(cid * DH, DH)])

    plsc.subcore_barrier()


def kernel(embed, edge_index, edge_values):
  packed = (edge_index[0] * 65536 + edge_index[1]).reshape(NBLK, W)
  wgt2 = edge_values.reshape(NBLK, W)
  embed_split = embed.reshape(N_NODES, NC, DH).transpose(1, 0, 2)

  mesh = plsc.VectorSubcoreMesh(core_axis_name="c", subcore_axis_name="s")
  out, _ = pl.kernel(
      _body,
      out_type=(
          jax.ShapeDtypeStruct((N_NODES, N_HOPS + 1, D_FEAT), jnp.float32),
          jax.ShapeDtypeStruct((N_HOPS - 1, NC, N_NODES, DH), jnp.float32),
      ),
      mesh=mesh,
      compiler_params=pltpu.CompilerParams(use_tc_tiling_on_sc=False),
      scratch_types=[
          pltpu.VMEM_SHARED((N_NODES, DH), jnp.float32),
          pltpu.VMEM((G * W, DH), jnp.float32),
          pltpu.VMEM((MAX_BLK, W), jnp.int32),
          pltpu.VMEM((MAX_BLK, W), jnp.float32),
          pltpu.VMEM((G, W), jnp.int32),
          pltpu.VMEM((G, W), jnp.int32),
          pltpu.VMEM((RCHUNK, DH), jnp.float32),
          pltpu.SemaphoreType.DMA((G,)),
      ],
  )(embed_split, packed, wgt2)
  return out


# speedup vs baseline: 9.8943x; 1.0193x over previous
import functools

import jax
import jax.numpy as jnp
from jax import lax
from jax.experimental import pallas as pl
from jax.experimental.pallas import tpu as pltpu
from jax.experimental.pallas import tpu_sc as plsc

N_NODES = 10000
N_EDGES = 320000
D_FEAT = 128
N_HOPS = 3
NC = 2
NS = 16
DH = D_FEAT // NC
W = 128
NBLK = N_EDGES // W
G = 4
BLK_PER_TILE = NBLK // NS
BLK_REM = NBLK % NS
MAX_BLK = BLK_PER_TILE + 1
N_GROUPS = (MAX_BLK + G - 1) // G
ROWS_PER_TILE = N_NODES // NS
RCHUNK = 125


def _body(embed_hbm, pk_hbm, wgt_hbm, out_hbm, tbl_hbm,
          acc, gbuf, pbig, wbig, colw, roww, zbuf, gsem):
  cid = lax.axis_index("c")
  sid = lax.axis_index("s")

  def _zero_row(i, _):
    for q in range(DH // 16):
      zbuf[i, pl.ds(q * 16, 16)] = jnp.zeros((16,), jnp.float32)
    return 0
  lax.fori_loop(0, RCHUNK, _zero_row, 0)

  base_blk = sid * BLK_PER_TILE + jnp.minimum(sid, BLK_REM)
  n_blk = BLK_PER_TILE + jnp.where(sid < BLK_REM, 1, 0)
  ld_base = jnp.minimum(base_blk, NBLK - MAX_BLK)
  off = base_blk - ld_base

  for kk in range(ROWS_PER_TILE // RCHUNK):
    r0 = sid * ROWS_PER_TILE + kk * RCHUNK
    pltpu.sync_copy(zbuf, acc.at[pl.ds(r0, RCHUNK)])

  pltpu.sync_copy(pk_hbm.at[pl.ds(ld_base, MAX_BLK)], pbig)
  pltpu.sync_copy(wgt_hbm.at[pl.ds(ld_base, MAX_BLK)], wbig)

  for kk in range(ROWS_PER_TILE // RCHUNK):
    r0 = sid * ROWS_PER_TILE + kk * RCHUNK
    stg = gbuf.at[pl.ds(0, RCHUNK)]
    pltpu.sync_copy(embed_hbm.at[pl.ds(r0, RCHUNK), pl.ds(cid * DH, DH)], stg)
    pltpu.sync_copy(stg, tbl_hbm.at[0].at[cid].at[pl.ds(r0, RCHUNK)])
    pltpu.sync_copy(
        stg, out_hbm.at[pl.ds(r0, RCHUNK), 0, pl.ds(cid * DH, DH)])

  plsc.subcore_barrier()

  def _unpack(jj, b):
    def _u16(k, _):
      p = pbig[off + jj, pl.ds(k * 16, 16)]
      colw[b, pl.ds(k * 16, 16)] = jnp.bitwise_and(p, 0xFFFF)
      roww[b, pl.ds(k * 16, 16)] = lax.shift_right_logical(p, 16)
      return 0
    lax.fori_loop(0, W // 16, _u16, 0)

  for hop in range(N_HOPS):
    tbl = tbl_hbm.at[hop]

    for b in range(G):
      _unpack(b, b)
      pltpu.async_copy(tbl.at[cid].at[colw.at[b]],
                       gbuf.at[pl.ds(b * W, W)], gsem.at[b])

    def _group(g, _, tbl=tbl):
      for b in range(G):
        jj = g * G + b

        @pl.when(jj < n_blk)
        def _():
          pltpu.make_async_copy(tbl.at[cid].at[colw.at[b]],
                                gbuf.at[pl.ds(b * W, W)], gsem.at[b]).wait()

          @plsc.parallel_loop(0, W // 16, unroll=2)
          def _scale16(k):
            wv = wbig[off + jj, pl.ds(k * 16, 16)]
            for l4 in range(4):
              vals, wss = [], []
              for l in range(4):
                e = b * W + k * 16 + l4 * 4 + l
                wss.append(jnp.broadcast_to(wv[l4 * 4 + l], (16,)))
                vals.append([gbuf[e, pl.ds(q * 16, 16)]
                             for q in range(DH // 16)])
              for l in range(4):
                e = b * W + k * 16 + l4 * 4 + l
                for q in range(DH // 16):
                  gbuf[e, pl.ds(q * 16, 16)] = vals[l][q] * wss[l]

          pltpu.sync_copy(gbuf.at[pl.ds(b * W, W)], acc.at[roww.at[b]],
                          add=True)

          @pl.when(jj + G < n_blk)
          def _():
            _unpack(jj + G, b)
            pltpu.async_copy(tbl.at[cid].at[colw.at[b]],
                             gbuf.at[pl.ds(b * W, W)], gsem.at[b])
      return 0

    lax.fori_loop(0, N_GROUPS, _group, 0)

    plsc.subcore_barrier()

    for kk in range(ROWS_PER_TILE // RCHUNK):
      r0 = sid * ROWS_PER_TILE + kk * RCHUNK
      stg = gbuf.at[pl.ds(0, RCHUNK)]
      pltpu.sync_copy(acc.at[pl.ds(r0, RCHUNK)], stg)
      pltpu.sync_copy(
          stg, out_hbm.at[pl.ds(r0, RCHUNK), hop + 1, pl.ds(cid * DH, DH)])
      if hop < N_HOPS - 1:
        pltpu.sync_copy(stg, tbl_hbm.at[hop + 1].at[cid].at[pl.ds(r0, RCHUNK)])
        pltpu.sync_copy(zbuf, acc.at[pl.ds(r0, RCHUNK)])

    plsc.subcore_barrier()


def kernel(embed, edge_index, edge_values):
  packed = (edge_index[0] * 65536 + edge_index[1]).reshape(NBLK, W)
  wgt2 = edge_values.reshape(NBLK, W)

  mesh = plsc.VectorSubcoreMesh(core_axis_name="c", subcore_axis_name="s")
  out, _ = pl.kernel(
      _body,
      out_type=(
          jax.ShapeDtypeStruct((N_NODES, N_HOPS + 1, D_FEAT), jnp.float32),
          jax.ShapeDtypeStruct((N_HOPS, NC, N_NODES, DH), jnp.float32),
      ),
      mesh=mesh,
      compiler_params=pltpu.CompilerParams(use_tc_tiling_on_sc=False),
      scratch_types=[
          pltpu.VMEM_SHARED((N_NODES, DH), jnp.float32),
          pltpu.VMEM((G * W, DH), jnp.float32),
          pltpu.VMEM((MAX_BLK, W), jnp.int32),
          pltpu.VMEM((MAX_BLK, W), jnp.float32),
          pltpu.VMEM((G, W), jnp.int32),
          pltpu.VMEM((G, W), jnp.int32),
          pltpu.VMEM((RCHUNK, DH), jnp.float32),
          pltpu.SemaphoreType.DMA((G,)),
      ],
  )(embed, packed, wgt2)
  return out


# speedup vs baseline: 12.2211x; 1.2352x over previous
import functools

import jax
import jax.numpy as jnp
from jax import lax
from jax.experimental import pallas as pl
from jax.experimental.pallas import tpu as pltpu
from jax.experimental.pallas import tpu_sc as plsc

N_NODES = 10000
N_EDGES = 320000
D_FEAT = 128
N_HOPS = 3
NC = 2
NS = 16
DH = D_FEAT // NC
W = 128
NBLK = N_EDGES // W
G = 4
BLK_PER_TILE = NBLK // NS
BLK_REM = NBLK % NS
MAX_BLK = BLK_PER_TILE + 1
N_GROUPS = (MAX_BLK + G - 1) // G
ROWS_PER_TILE = N_NODES // NS
RCHUNK = 125


def _body(embed_hbm, pk_hbm, wgt_hbm, out_hbm, tbl_hbm,
          acc, gbuf, pbig, wbig, colw, roww, zbuf, gsem):
  cid = lax.axis_index("c")
  sid = lax.axis_index("s")

  def _zero_row(i, _):
    for q in range(DH // 16):
      zbuf[i, pl.ds(q * 16, 16)] = jnp.zeros((16,), jnp.float32)
    return 0
  lax.fori_loop(0, RCHUNK, _zero_row, 0)

  base_blk = sid * BLK_PER_TILE + jnp.minimum(sid, BLK_REM)
  n_blk = BLK_PER_TILE + jnp.where(sid < BLK_REM, 1, 0)
  ld_base = jnp.minimum(base_blk, NBLK - MAX_BLK)
  off = base_blk - ld_base

  for kk in range(ROWS_PER_TILE // RCHUNK):
    r0 = sid * ROWS_PER_TILE + kk * RCHUNK
    pltpu.sync_copy(zbuf, acc.at[pl.ds(r0, RCHUNK)])

  pltpu.sync_copy(pk_hbm.at[pl.ds(ld_base, MAX_BLK)], pbig)
  pltpu.sync_copy(wgt_hbm.at[pl.ds(ld_base, MAX_BLK)], wbig)

  for kk in range(ROWS_PER_TILE // RCHUNK):
    r0 = sid * ROWS_PER_TILE + kk * RCHUNK
    stg = gbuf.at[pl.ds(0, RCHUNK)]
    pltpu.sync_copy(embed_hbm.at[pl.ds(r0, RCHUNK), pl.ds(cid * DH, DH)], stg)
    pltpu.sync_copy(stg, tbl_hbm.at[0].at[cid].at[pl.ds(r0, RCHUNK)])
    pltpu.sync_copy(
        stg, out_hbm.at[pl.ds(r0, RCHUNK), 0, pl.ds(cid * DH, DH)])

  plsc.subcore_barrier()

  def _unpack(jj, b):
    def _u16(k, _):
      p = pbig[off + jj, pl.ds(k * 16, 16)]
      colw[b, pl.ds(k * 16, 16)] = jnp.bitwise_and(p, 0xFFFF)
      roww[b, pl.ds(k * 16, 16)] = lax.shift_right_logical(p, 16)
      return 0
    lax.fori_loop(0, W // 16, _u16, 0)

  for hop in range(N_HOPS):
    tbl = tbl_hbm.at[hop]

    for b in range(G):
      _unpack(b, b)
      pltpu.async_copy(tbl.at[cid].at[colw.at[b]],
                       gbuf.at[pl.ds(b * W, W)], gsem.at[b])

    def _group(g, _, tbl=tbl):
      for b in range(G):
        jj = g * G + b

        @pl.when(jj < n_blk)
        def _():
          pltpu.make_async_copy(tbl.at[cid].at[colw.at[b]],
                                gbuf.at[pl.ds(b * W, W)], gsem.at[b]).wait()

          @plsc.parallel_loop(0, W // 16, unroll=4)
          def _scale16(k):
            wv = wbig[off + jj, pl.ds(k * 16, 16)]
            for l4 in range(4):
              vals, wss = [], []
              for l in range(4):
                e = b * W + k * 16 + l4 * 4 + l
                wss.append(jnp.broadcast_to(wv[l4 * 4 + l], (16,)))
                vals.append([gbuf[e, pl.ds(q * 16, 16)]
                             for q in range(DH // 16)])
              for l in range(4):
                e = b * W + k * 16 + l4 * 4 + l
                for q in range(DH // 16):
                  gbuf[e, pl.ds(q * 16, 16)] = vals[l][q] * wss[l]

          pltpu.sync_copy(gbuf.at[pl.ds(b * W, W)], acc.at[roww.at[b]],
                          add=True)

          @pl.when(jj + G < n_blk)
          def _():
            _unpack(jj + G, b)
            pltpu.async_copy(tbl.at[cid].at[colw.at[b]],
                             gbuf.at[pl.ds(b * W, W)], gsem.at[b])
      return 0

    lax.fori_loop(0, N_GROUPS, _group, 0)

    plsc.subcore_barrier()

    for kk in range(ROWS_PER_TILE // RCHUNK):
      r0 = sid * ROWS_PER_TILE + kk * RCHUNK
      stg = gbuf.at[pl.ds(0, RCHUNK)]
      pltpu.sync_copy(acc.at[pl.ds(r0, RCHUNK)], stg)
      pltpu.sync_copy(
          stg, out_hbm.at[pl.ds(r0, RCHUNK), hop + 1, pl.ds(cid * DH, DH)])
      if hop < N_HOPS - 1:
        pltpu.sync_copy(stg, tbl_hbm.at[hop + 1].at[cid].at[pl.ds(r0, RCHUNK)])
        pltpu.sync_copy(zbuf, acc.at[pl.ds(r0, RCHUNK)])

    plsc.subcore_barrier()


def kernel(embed, edge_index, edge_values):
  packed = (edge_index[0] * 65536 + edge_index[1]).reshape(NBLK, W)
  wgt2 = edge_values.reshape(NBLK, W)

  mesh = plsc.VectorSubcoreMesh(core_axis_name="c", subcore_axis_name="s")
  out, _ = pl.kernel(
      _body,
      out_type=(
          jax.ShapeDtypeStruct((N_NODES, N_HOPS + 1, D_FEAT), jnp.float32),
          jax.ShapeDtypeStruct((N_HOPS, NC, N_NODES, DH), jnp.float32),
      ),
      mesh=mesh,
      compiler_params=pltpu.CompilerParams(use_tc_tiling_on_sc=False),
      scratch_types=[
          pltpu.VMEM_SHARED((N_NODES, DH), jnp.float32),
          pltpu.VMEM((G * W, DH), jnp.float32),
          pltpu.VMEM((MAX_BLK, W), jnp.int32),
          pltpu.VMEM((MAX_BLK, W), jnp.float32),
          pltpu.VMEM((G, W), jnp.int32),
          pltpu.VMEM((G, W), jnp.int32),
          pltpu.VMEM((RCHUNK, DH), jnp.float32),
          pltpu.SemaphoreType.DMA((G,)),
      ],
  )(embed, packed, wgt2)
  return out
